# Initial kernel scaffold; baseline (speedup 1.0000x reference)
import jax
import jax.numpy as jnp
from jax.experimental import pallas as pl


def kernel(positive_triplets, negative_triplets, entity_emb, relation_emb):
    raise NotImplementedError("write your pallas kernel here")


# speedup vs baseline: 1.0831x; 1.0831x over previous
import functools

import jax
import jax.numpy as jnp
from jax import lax
from jax.experimental import pallas as pl
from jax.experimental.pallas import tpu as pltpu
from jax.experimental.pallas import tpu_sc as plsc

_ENTITY_SIZE = 1000000
_EMBED_DIM = 64
_MARGIN = 1.0

_L = 16
_NQ = _EMBED_DIM // _L
_CHUNK = 128
_SLOT = 2 * _L


def _rsqrt_nr(x):
    i = lax.bitcast_convert_type(x, jnp.int32)
    y = lax.bitcast_convert_type(jnp.int32(0x5F3759DF) - (i >> 1), jnp.float32)
    for _ in range(3):
        y = y * (1.5 - 0.5 * x * y * y)
    return y


def _bcast_i32(s):
    return jnp.full((_L,), 0, jnp.int32) + s


def _lane_sum_splat(q, scr, s0):
    scr[pl.ds(s0, _L)] = q
    scr[pl.ds(s0 + _L, _L)] = q
    s = q + scr[pl.ds(s0 + 4, _L)] + scr[pl.ds(s0 + 8, _L)] \
        + scr[pl.ds(s0 + 12, _L)]
    scr[pl.ds(s0, _L)] = s
    scr[pl.ds(s0 + _L, _L)] = s
    return s + scr[pl.ds(s0 + 1, _L)] + scr[pl.ds(s0 + 2, _L)] \
        + scr[pl.ds(s0 + 3, _L)]


def _triplet_distance(hrow_v, rrow_v, trow_v, eh_g, et_g, scr, i, j):
    h = [hrow_v[i, pl.ds(k * _L, _L)] for k in range(_NQ)]
    r = [rrow_v[i, pl.ds(k * _L, _L)] for k in range(_NQ)]
    t = [trow_v[i, pl.ds(k * _L, _L)] for k in range(_NQ)]

    hh_p = h[0] * h[0]
    tt_p = t[0] * t[0]
    for k in range(1, _NQ):
        hh_p = hh_p + h[k] * h[k]
        tt_p = tt_p + t[k] * t[k]
    hh = _lane_sum_splat(hh_p, scr, (2 * j) * _SLOT)
    tt = _lane_sum_splat(tt_p, scr, (2 * j + 1) * _SLOT)

    eh = jnp.full((_L,), 0.0, jnp.float32) + eh_g[j]
    et = jnp.full((_L,), 0.0, jnp.float32) + et_g[j]
    a = _rsqrt_nr(hh)
    a = a + eh * (1.0 - a)
    b = _rsqrt_nr(tt)
    b = b + et * (1.0 - b)

    ss_p = jnp.zeros((_L,), jnp.float32)
    for k in range(_NQ):
        s = h[k] * a + r[k] - t[k] * b
        ss_p = ss_p + s * s
    d2 = _lane_sum_splat(ss_p, scr, (2 * j) * _SLOT)
    return jnp.where(d2 > 0.0, d2 * _rsqrt_nr(d2),
                     jnp.zeros((_L,), jnp.float32))


def _transe_sc(hidx_all, ridx_all, tidx_all, ent_hbm, rel_hbm, batch):
    info = plsc.get_sparse_core_info()
    nw = info.num_cores * info.num_subcores
    per_w = batch // nw
    n_chunks = per_w // _CHUNK
    mesh = plsc.VectorSubcoreMesh(core_axis_name="c", subcore_axis_name="s")

    @functools.partial(
        pl.kernel,
        mesh=mesh,
        out_type=jax.ShapeDtypeStruct((batch,), jnp.float32),
        compiler_params=pltpu.CompilerParams(use_tc_tiling_on_sc=False),
        scratch_types=[
            pltpu.VMEM((_CHUNK,), jnp.int32),
            pltpu.VMEM((_CHUNK,), jnp.int32),
            pltpu.VMEM((_CHUNK,), jnp.int32),
            pltpu.VMEM((_CHUNK, _EMBED_DIM), jnp.float32),
            pltpu.VMEM((_CHUNK, _EMBED_DIM), jnp.float32),
            pltpu.VMEM((_CHUNK, _EMBED_DIM), jnp.float32),
            pltpu.VMEM((2 * _L * _SLOT,), jnp.float32),
            pltpu.VMEM((2 * per_w,), jnp.float32),
            pltpu.VMEM((per_w,), jnp.float32),
            pltpu.SemaphoreType.DMA,
        ],
    )
    def k(hidx_h, ridx_h, tidx_h, ent_h, rel_h, out_h,
          hidx_v, ridx_v, tidx_v, hrow_v, rrow_v, trow_v,
          scr_v, dist_v, loss_v, sem):
        wid = lax.axis_index("s") * info.num_cores + lax.axis_index("c")
        wbase = wid * per_w
        lanes = lax.iota(jnp.int32, _L)

        def chunk_body(c, carry):
            p = c // n_chunks
            cc = c - p * n_chunks
            src = p * batch + wbase + cc * _CHUNK
            pltpu.sync_copy(hidx_h.at[pl.ds(src, _CHUNK)], hidx_v)
            pltpu.sync_copy(ridx_h.at[pl.ds(src, _CHUNK)], ridx_v)
            pltpu.sync_copy(tidx_h.at[pl.ds(src, _CHUNK)], tidx_v)
            cp_h = pltpu.async_copy(ent_h.at[hidx_v], hrow_v, sem)
            cp_r = pltpu.async_copy(rel_h.at[ridx_v], rrow_v, sem)
            cp_t = pltpu.async_copy(ent_h.at[tidx_v], trow_v, sem)
            cp_h.wait()
            cp_r.wait()
            cp_t.wait()

            dbase = p * per_w + cc * _CHUNK

            def group_body(g, carry2):
                gb = g * _L
                last = jnp.full((_L,), _ENTITY_SIZE - 1, jnp.int32)
                onef = jnp.ones((_L,), jnp.float32)
                zerof = jnp.zeros((_L,), jnp.float32)
                eh_g = jnp.where(hidx_v[pl.ds(gb, _L)] == last, onef, zerof)
                et_g = jnp.where(tidx_v[pl.ds(gb, _L)] == last, onef, zerof)
                d_acc = jnp.zeros((_L,), jnp.float32)
                for j in range(_L):
                    d = _triplet_distance(hrow_v, rrow_v, trow_v,
                                          eh_g, et_g, scr_v, gb + j, j)
                    d_acc = jnp.where(lanes == j, d, d_acc)
                dist_v[pl.ds(dbase + gb, _L)] = d_acc
                return carry2

            lax.fori_loop(0, _CHUNK // _L, group_body, 0)
            return carry

        lax.fori_loop(0, 2 * n_chunks, chunk_body, 0)

        def loss_body(g, carry):
            gb = g * _L
            dp = dist_v[pl.ds(gb, _L)]
            dn = dist_v[pl.ds(per_w + gb, _L)]
            loss_v[pl.ds(gb, _L)] = jnp.maximum(dp - dn + _MARGIN, 0.0)
            return carry

        lax.fori_loop(0, per_w // _L, loss_body, 0)
        pltpu.sync_copy(loss_v, out_h.at[pl.ds(wbase, per_w)])

    return k(hidx_all, ridx_all, tidx_all, ent_hbm, rel_hbm)


def kernel(positive_triplets, negative_triplets, entity_emb, relation_emb):
    batch = positive_triplets.shape[0]
    cols = jnp.concatenate(
        [positive_triplets.astype(jnp.int32),
         negative_triplets.astype(jnp.int32)], axis=0).T
    return _transe_sc(cols[0], cols[1], cols[2],
                      entity_emb, relation_emb, batch)
